# Initial kernel scaffold; baseline (speedup 1.0000x reference)
import jax
import jax.numpy as jnp
from jax.experimental import pallas as pl


def kernel(x, edge_index, W1, b1, W2, b2):
    raise NotImplementedError("write your pallas kernel here")


# speedup vs baseline: 5.8325x; 5.8325x over previous
import functools

import jax
import jax.numpy as jnp
from jax import lax
from jax.experimental import pallas as pl
from jax.experimental.pallas import tpu as pltpu
from jax.experimental.pallas import tpu_sc as plsc

N = 10000
E = 160000
D = 256
DH = 128
N2 = 10240
NC = 2
NS = 16
NW = NC * NS
CH = 128
NCHUNK = E // CH
ROWS_PER_TILE = N2 // NS

_mesh = plsc.VectorSubcoreMesh(
    core_axis_name="c", subcore_axis_name="s", num_cores=NC, num_subcores=NS
)


def _wid_and_chunks():
    c = lax.axis_index("c")
    s = lax.axis_index("s")
    wid = s * NC + c
    nch = jnp.where(wid < NCHUNK - (NCHUNK // NW) * NW, NCHUNK // NW + 1,
                    NCHUNK // NW)
    return c, s, wid, nch


def _deg_body(src_hbm, dst_hbm, out_hbm, srcv, dstv, dego, degi):
    c, s, wid, nch = _wid_and_chunks()
    zeros16 = jnp.zeros((16,), jnp.float32)
    ones16 = jnp.ones((16,), jnp.float32)

    def zero_body(i, _):
        dego[pl.ds(i * 16, 16)] = zeros16
        degi[pl.ds(i * 16, 16)] = zeros16
        return 0

    lax.fori_loop(0, N2 // 16, zero_body, 0)

    def chunk_body(i, _):
        ch = wid + NW * i
        pltpu.sync_copy(src_hbm.at[ch], srcv)
        pltpu.sync_copy(dst_hbm.at[ch], dstv)
        for j in range(CH // 16):
            si = srcv[pl.ds(16 * j, 16)]
            plsc.addupdate_scatter(dego, [si], ones16)
            di = dstv[pl.ds(16 * j, 16)]
            plsc.addupdate_scatter(degi, [di], ones16)
        return 0

    lax.fori_loop(0, nch, chunk_body, 0)
    pltpu.sync_copy(dego, out_hbm.at[c, s, 0])
    pltpu.sync_copy(degi, out_hbm.at[c, s, 1])


_deg = pl.kernel(
    _deg_body,
    out_type=jax.ShapeDtypeStruct((NC, NS, 2, N2), jnp.float32),
    mesh=_mesh,
    scratch_types=[
        pltpu.VMEM((CH,), jnp.int32),
        pltpu.VMEM((CH,), jnp.int32),
        pltpu.VMEM((N2,), jnp.float32),
        pltpu.VMEM((N2,), jnp.float32),
    ],
    compiler_params=pltpu.CompilerParams(needs_layout_passes=False),
)


def _agg_body(hs_hbm, src_hbm, dst_hbm, zrows_hbm, out_hbm,
              srcv, dstv, rows, agg_sh, sem):
    c = lax.axis_index("c")
    s = lax.axis_index("s")
    nch = jnp.where(s < NCHUNK - (NCHUNK // NS) * NS, NCHUNK // NS + 1,
                    NCHUNK // NS)
    pltpu.sync_copy(zrows_hbm, agg_sh.at[pl.ds(s * ROWS_PER_TILE,
                                               ROWS_PER_TILE)])
    plsc.subcore_barrier()

    off = c * N2

    def chunk_body(i, _):
        ch = s + NS * i
        pltpu.sync_copy(src_hbm.at[ch], srcv)
        pltpu.sync_copy(dst_hbm.at[ch], dstv)
        for j in range(CH // 16):
            sl = pl.ds(16 * j, 16)
            srcv[sl] = srcv[sl] + off
        pltpu.async_copy(hs_hbm.at[srcv], rows, sem).wait()
        pltpu.sync_copy(rows, agg_sh.at[dstv], add=True)
        return 0

    lax.fori_loop(0, nch, chunk_body, 0)
    plsc.subcore_barrier()
    pltpu.sync_copy(
        agg_sh.at[pl.ds(s * ROWS_PER_TILE, ROWS_PER_TILE)],
        out_hbm.at[pl.ds(off + s * ROWS_PER_TILE, ROWS_PER_TILE)],
    )


_agg = pl.kernel(
    _agg_body,
    out_type=jax.ShapeDtypeStruct((NC * N2, DH), jnp.float32),
    mesh=_mesh,
    scratch_types=[
        pltpu.VMEM((CH,), jnp.int32),
        pltpu.VMEM((CH,), jnp.int32),
        pltpu.VMEM((CH, DH), jnp.float32),
        pltpu.VMEM_SHARED((N2, DH), jnp.float32),
        pltpu.SemaphoreType.DMA,
    ],
    compiler_params=pltpu.CompilerParams(needs_layout_passes=False),
)


_HI = jax.lax.Precision.HIGHEST
_BN = 1024
_NB = N2 // _BN


def _norms_body(degp_ref, ns_ref, nd_ref):
    d = jnp.sum(degp_ref[...], axis=0)
    bnl = d.shape[1]
    ii = lax.broadcasted_iota(jnp.int32, (bnl, bnl), 0)
    jj = lax.broadcasted_iota(jnp.int32, (bnl, bnl), 1)
    ident = jnp.where(ii == jj, 1.0, 0.0)
    cols = lax.dot_general(ident, d, (((1,), (1,)), ((), ())), precision=_HI)
    deg_out = cols[:, 0:1]
    deg_in = cols[:, 1:2]
    ns_ref[...] = jnp.where(deg_out > 0.0,
                            lax.rsqrt(jnp.maximum(deg_out, 1e-12)), 0.0)
    nd_ref[...] = jnp.where(deg_in > 0.0,
                            lax.rsqrt(jnp.maximum(deg_in, 1e-12)), 0.0)


_NORM_BN = 256


def _norms(degp):
    return pl.pallas_call(
        _norms_body,
        grid=(N2 // _NORM_BN,),
        in_specs=[pl.BlockSpec((NW, 2, _NORM_BN), lambda b: (0, 0, b))],
        out_specs=[
            pl.BlockSpec((_NORM_BN, 1), lambda b: (b, 0)),
            pl.BlockSpec((_NORM_BN, 1), lambda b: (b, 0)),
        ],
        out_shape=[
            jax.ShapeDtypeStruct((N2, 1), jnp.float32),
            jax.ShapeDtypeStruct((N2, 1), jnp.float32),
        ],
    )(degp)


def _mm1_body(x_ref, w_ref, ns_ref, p0_ref, p1_ref):
    h = jnp.dot(x_ref[...], w_ref[...], precision=_HI)
    hs = h * ns_ref[...]
    p0_ref[...] = hs[:, :DH]
    p1_ref[...] = hs[:, DH:]


def _mm1(xp, W1, ns):
    return pl.pallas_call(
        _mm1_body,
        grid=(_NB,),
        in_specs=[
            pl.BlockSpec((_BN, D), lambda b: (b, 0)),
            pl.BlockSpec((D, D), lambda b: (0, 0)),
            pl.BlockSpec((_BN, 1), lambda b: (b, 0)),
        ],
        out_specs=[
            pl.BlockSpec((_BN, DH), lambda b: (b, 0)),
            pl.BlockSpec((_BN, DH), lambda b: (b, 0)),
        ],
        out_shape=[
            jax.ShapeDtypeStruct((N2, DH), jnp.float32),
            jax.ShapeDtypeStruct((N2, DH), jnp.float32),
        ],
    )(xp, W1, ns)


def _mm2_body(a0_ref, a1_ref, nd_ref, b1_ref, w_ref, ns_ref, p0_ref, p1_ref):
    a = jnp.concatenate([a0_ref[...], a1_ref[...]], axis=1)
    t = jnp.maximum(a * nd_ref[...] + b1_ref[...], 0.0)
    h = jnp.dot(t, w_ref[...], precision=_HI)
    hs = h * ns_ref[...]
    p0_ref[...] = hs[:, :DH]
    p1_ref[...] = hs[:, DH:]


def _mm2(agg1, nd, b1, W2, ns):
    return pl.pallas_call(
        _mm2_body,
        grid=(_NB,),
        in_specs=[
            pl.BlockSpec((_BN, DH), lambda b: (b, 0)),
            pl.BlockSpec((_BN, DH), lambda b: (b + _NB, 0)),
            pl.BlockSpec((_BN, 1), lambda b: (b, 0)),
            pl.BlockSpec((1, D), lambda b: (0, 0)),
            pl.BlockSpec((D, D), lambda b: (0, 0)),
            pl.BlockSpec((_BN, 1), lambda b: (b, 0)),
        ],
        out_specs=[
            pl.BlockSpec((_BN, DH), lambda b: (b, 0)),
            pl.BlockSpec((_BN, DH), lambda b: (b, 0)),
        ],
        out_shape=[
            jax.ShapeDtypeStruct((N2, DH), jnp.float32),
            jax.ShapeDtypeStruct((N2, DH), jnp.float32),
        ],
    )(agg1, agg1, nd, b1, W2, ns)


def _final_body(a0_ref, a1_ref, nd_ref, b2_ref, out_ref):
    a = jnp.concatenate([a0_ref[...], a1_ref[...]], axis=1)
    out_ref[...] = a * nd_ref[...] + b2_ref[...]


def _final(agg2, nd, b2):
    return pl.pallas_call(
        _final_body,
        grid=(_NB,),
        in_specs=[
            pl.BlockSpec((_BN, DH), lambda b: (b, 0)),
            pl.BlockSpec((_BN, DH), lambda b: (b + _NB, 0)),
            pl.BlockSpec((_BN, 1), lambda b: (b, 0)),
            pl.BlockSpec((1, D), lambda b: (0, 0)),
        ],
        out_specs=pl.BlockSpec((_BN, D), lambda b: (b, 0)),
        out_shape=jax.ShapeDtypeStruct((N2, D), jnp.float32),
    )(agg2, agg2, nd, b2)


@jax.jit
def kernel(x, edge_index, W1, b1, W2, b2):
    srcd = edge_index[0].reshape(NCHUNK, CH)
    dstd = edge_index[1].reshape(NCHUNK, CH)

    degp = _deg(srcd, dstd).reshape(NC * NS, 2, N2)
    ns, nd = _norms(degp)

    xp = jnp.pad(x, ((0, N2 - N), (0, 0)))
    p0, p1 = _mm1(xp, W1, ns)
    hs1 = jnp.concatenate([p0, p1], axis=0)
    zrows = jnp.zeros((ROWS_PER_TILE, DH), jnp.float32)
    agg1 = _agg(hs1, srcd, dstd, zrows)

    p0, p1 = _mm2(agg1, nd, b1.reshape(1, D), W2, ns)
    hs2 = jnp.concatenate([p0, p1], axis=0)
    agg2 = _agg(hs2, srcd, dstd, zrows)

    out = _final(agg2, nd, b2.reshape(1, D))
    return out[:N]
